# baseline (speedup 1.0000x reference)
import functools

import jax
import jax.numpy as jnp
from jax import lax
from jax.experimental import pallas as pl
from jax.experimental.pallas import tpu as pltpu
from jax.experimental.pallas import tpu_sc as plsc

B = 4096
NNZ = 256
M = 2048
L = 16

NC = 2
NS = 16
NW = NC * NS
ROWS = B // NW
G = 8
NG = ROWS // G
NQ = NNZ // L


def _body(idx_hbm, val_hbm, out_hbm, idx_v, val_v, dense0, dense1,
          sem0, sem1):
    c = lax.axis_index("c")
    s = lax.axis_index("s")
    wid = s * NC + c
    base = wid * ROWS
    pltpu.sync_copy(idx_hbm.at[pl.ds(base, ROWS)], idx_v)
    pltpu.sync_copy(val_hbm.at[pl.ds(base, ROWS)], val_v)

    zeros16 = jnp.zeros((L,), jnp.float32)
    bufs = (dense0, dense1)
    sems = (sem0, sem1)

    def fill_group(b, g0):
        dense = bufs[b]

        @plsc.parallel_loop(0, G * M // L, unroll=16)
        def _zero(i):
            dense[pl.ds(i * L, L)] = zeros16

        @plsc.parallel_loop(0, G, unroll=2)
        def _scatter_row(g):
            r = g0 + g
            off = g * M
            for q in range(NQ):
                idx16 = idx_v[r, pl.ds(q * L, L)] + off
                val16 = val_v[r, pl.ds(q * L, L)]
                plsc.addupdate_scatter(dense, [idx16], val16)

        pltpu.async_copy(
            dense, out_hbm.at[pl.ds((base + g0) * M, G * M)], sems[b]
        )

    def wait_group(b, g0):
        pltpu.make_async_copy(
            bufs[b], out_hbm.at[pl.ds((base + g0) * M, G * M)], sems[b]
        ).wait()

    fill_group(0, 0)
    fill_group(1, G)

    def pair(p, carry):
        g0 = 2 * p * G
        wait_group(0, g0 - 2 * G)
        fill_group(0, g0)
        wait_group(1, g0 - G)
        fill_group(1, g0 + G)
        return carry

    lax.fori_loop(1, NG // 2, pair, 0)
    wait_group(0, (NG - 2) * G)
    wait_group(1, (NG - 1) * G)


_sc_call = functools.partial(
    pl.kernel,
    mesh=plsc.VectorSubcoreMesh(core_axis_name="c", subcore_axis_name="s"),
    out_type=jax.ShapeDtypeStruct((B * M,), jnp.float32),
    compiler_params=pltpu.CompilerParams(needs_layout_passes=False),
    scratch_types=[
        pltpu.VMEM((ROWS, NNZ), jnp.int32),
        pltpu.VMEM((ROWS, NNZ), jnp.float32),
        pltpu.VMEM((G * M,), jnp.float32),
        pltpu.VMEM((G * M,), jnp.float32),
        pltpu.SemaphoreType.DMA,
        pltpu.SemaphoreType.DMA,
    ],
)(_body)


def kernel(indices, values):
    return _sc_call(indices, values).reshape(B, M)


# speedup vs baseline: 1.7706x; 1.7706x over previous
import functools

import jax
import jax.numpy as jnp
from jax import lax
from jax.experimental import pallas as pl
from jax.experimental.pallas import tpu as pltpu
from jax.experimental.pallas import tpu_sc as plsc

B = 4096
NNZ = 256
M = 2048
L = 16

NC = 2
NS = 16
NW = NC * NS
ROWS = B // NW
G = 8
NG = ROWS // G
NQ = NNZ // L


def _body(idx_hbm, val_hbm, out_hbm, idx_v, val_v, dense0, dense1,
          sem0, sem1):
    c = lax.axis_index("c")
    s = lax.axis_index("s")
    wid = s * NC + c
    base = wid * ROWS
    pltpu.sync_copy(idx_hbm.at[pl.ds(base, ROWS)], idx_v)
    pltpu.sync_copy(val_hbm.at[pl.ds(base, ROWS)], val_v)

    zeros16 = jnp.zeros((L,), jnp.float32)
    bufs = (dense0, dense1)
    sems = (sem0, sem1)

    def fill_group(b, g0):
        dense = bufs[b]

        @plsc.parallel_loop(0, G * M // L, unroll=16)
        def _zero(i):
            dense[i // (M // L), pl.ds((i % (M // L)) * L, L)] = zeros16

        @plsc.parallel_loop(0, G, unroll=2)
        def _scatter_row(g):
            r = g0 + g
            gvec = jnp.full((L,), 0, jnp.int32) + g
            for q in range(NQ):
                idx16 = idx_v[r, pl.ds(q * L, L)]
                val16 = val_v[r, pl.ds(q * L, L)]
                plsc.addupdate_scatter(dense, [gvec, idx16], val16)

        pltpu.async_copy(
            dense, out_hbm.at[pl.ds(base + g0, G)], sems[b]
        )

    def wait_group(b, g0):
        pltpu.make_async_copy(
            bufs[b], out_hbm.at[pl.ds(base + g0, G)], sems[b]
        ).wait()

    fill_group(0, 0)
    fill_group(1, G)

    def pair(p, carry):
        g0 = 2 * p * G
        wait_group(0, g0 - 2 * G)
        fill_group(0, g0)
        wait_group(1, g0 - G)
        fill_group(1, g0 + G)
        return carry

    lax.fori_loop(1, NG // 2, pair, 0)
    wait_group(0, (NG - 2) * G)
    wait_group(1, (NG - 1) * G)


_sc_call = functools.partial(
    pl.kernel,
    mesh=plsc.VectorSubcoreMesh(core_axis_name="c", subcore_axis_name="s"),
    out_type=jax.ShapeDtypeStruct((B, M), jnp.float32),
    compiler_params=pltpu.CompilerParams(needs_layout_passes=False),
    scratch_types=[
        pltpu.VMEM((ROWS, NNZ), jnp.int32),
        pltpu.VMEM((ROWS, NNZ), jnp.float32),
        pltpu.VMEM((G, M), jnp.float32),
        pltpu.VMEM((G, M), jnp.float32),
        pltpu.SemaphoreType.DMA,
        pltpu.SemaphoreType.DMA,
    ],
)(_body)


def kernel(indices, values):
    return _sc_call(indices, values)
